# baseline (speedup 1.0000x reference)
import jax
import jax.numpy as jnp
from jax import lax
from jax.experimental import pallas as pl
from jax.experimental.pallas import tpu as pltpu
from jax.experimental.pallas import tpu_sc as plsc

N = 10000
E = 160000
D = 256
H = 8
DH = D // H

NC = 2
NS = 16
L = 16
HC = H // NC
DC = D // NC

RP = 10016
STRIPE = RP // NS
TAIL = N - (NS - 1) * STRIPE
CHUNK = 80
EPT = 10000
NCHUNK = EPT // CHUNK
EPAD = EPT * NS
TRASH_DST = 10008

BN = 1000
GB = N // BN


def _proj_body(x_ref, wp_ref, as_ref, ad_ref, ht_ref, es_ref, ed_ref):
    ht = jnp.dot(x_ref[...], wp_ref[0], preferred_element_type=jnp.float32)
    ht_ref[...] = ht
    es_ref[...] = jnp.dot(ht, as_ref[0], preferred_element_type=jnp.float32)
    ed_ref[...] = jnp.dot(ht, ad_ref[0], preferred_element_type=jnp.float32)


def _project(x, Wp, As, Ad):
    return pl.pallas_call(
        _proj_body,
        grid=(GB, NC),
        in_specs=[
            pl.BlockSpec((BN, D), lambda i, c: (i, 0)),
            pl.BlockSpec((1, D, DC), lambda i, c: (c, 0, 0)),
            pl.BlockSpec((1, DC, L), lambda i, c: (c, 0, 0)),
            pl.BlockSpec((1, DC, L), lambda i, c: (c, 0, 0)),
        ],
        out_specs=[
            pl.BlockSpec((BN, DC), lambda i, c: (c * GB + i, 0)),
            pl.BlockSpec((BN, L), lambda i, c: (c * GB + i, 0)),
            pl.BlockSpec((BN, L), lambda i, c: (c * GB + i, 0)),
        ],
        out_shape=[
            jax.ShapeDtypeStruct((NC * N, DC), jnp.float32),
            jax.ShapeDtypeStruct((NC * N, L), jnp.float32),
            jax.ShapeDtypeStruct((NC * N, L), jnp.float32),
        ],
    )(x, Wp, As, Ad)


def _gat_sc_body(ht_hbm, es_hbm, ed_hbm, esrc_hbm, edst_hbm, acc_out, den_out,
                 csrc0, csrc1, cdst0, cdst1, gsrc0, gsrc1, gdst0, gdst1,
                 sidx0, sidx1, esb0, esb1, edb0, edb1, wb0, wb1, hb0, hb1,
                 acc_sh, den_sh,
                 s_src0, s_src1, s_dst0, s_dst1, s_es0, s_es1, s_ed0, s_ed1,
                 s_ht0, s_ht1, s_de0, s_de1, s_ac0, s_ac1):
    cid = lax.axis_index("c")
    sid = lax.axis_index("s")
    base_t = (cid * N).astype(jnp.int32)
    base_tv = jnp.broadcast_to(base_t, (L,))
    nclampv = jnp.full((L,), N - 1, jnp.int32)
    zf = jnp.zeros((L,), jnp.float32)
    ebase = sid * EPT

    slot0 = (csrc0, cdst0, gsrc0, gdst0, sidx0, esb0, edb0, wb0, hb0,
             s_src0, s_dst0, s_es0, s_ed0, s_ht0, s_de0, s_ac0)
    slot1 = (csrc1, cdst1, gsrc1, gdst1, sidx1, esb1, edb1, wb1, hb1,
             s_src1, s_dst1, s_es1, s_ed1, s_ht1, s_de1, s_ac1)

    rbase = sid * STRIPE

    def issue_edges(k, sl):
        csrc, cdst = sl[0], sl[1]
        s_src, s_dst = sl[9], sl[10]
        eb = ebase + k * CHUNK
        pltpu.async_copy(esrc_hbm.at[pl.ds(eb, CHUNK)], csrc, s_src)
        pltpu.async_copy(edst_hbm.at[pl.ds(eb, CHUNK)], cdst, s_dst)

    def wait_edges(k, sl):
        csrc, cdst = sl[0], sl[1]
        s_src, s_dst = sl[9], sl[10]
        eb = ebase + k * CHUNK
        pltpu.make_async_copy(esrc_hbm.at[pl.ds(eb, CHUNK)], csrc, s_src).wait()
        pltpu.make_async_copy(edst_hbm.at[pl.ds(eb, CHUNK)], cdst, s_dst).wait()

    def prep_idx(sl):
        csrc, cdst, gsrc, gdst, sidx = sl[0], sl[1], sl[2], sl[3], sl[4]
        for q in range(CHUNK // L):
            sv = csrc[pl.ds(q * L, L)]
            gsrc[pl.ds(q * L, L)] = sv + base_tv
            dv = cdst[pl.ds(q * L, L)]
            sidx[pl.ds(q * L, L)] = dv
            gdst[pl.ds(q * L, L)] = jnp.minimum(dv, nclampv) + base_tv

    def issue_gathers(sl):
        gsrc, gdst, esb, edb, hb = sl[2], sl[3], sl[5], sl[6], sl[8]
        s_es, s_ed, s_ht = sl[11], sl[12], sl[13]
        pltpu.async_copy(es_hbm.at[gsrc], esb, s_es)
        pltpu.async_copy(ed_hbm.at[gdst], edb, s_ed)
        pltpu.async_copy(ht_hbm.at[gsrc], hb, s_ht)

    def wait_eled(sl):
        gsrc, gdst, esb, edb = sl[2], sl[3], sl[5], sl[6]
        s_es, s_ed = sl[11], sl[12]
        pltpu.make_async_copy(es_hbm.at[gsrc], esb, s_es).wait()
        pltpu.make_async_copy(ed_hbm.at[gdst], edb, s_ed).wait()

    def compute_and_scatter(sl):
        gsrc, sidx, esb, edb, wb, hb = sl[2], sl[4], sl[5], sl[6], sl[7], sl[8]
        s_ht, s_de, s_ac = sl[13], sl[14], sl[15]

        def wcomp(i, _):
            for t in range(2):
                i2 = 2 * i + t
                s = esb[i2, pl.ds(0, L)] + edb[i2, pl.ds(0, L)]
                lr = jnp.where(s > 0, s, 0.2 * s)
                wb[i2, pl.ds(0, L)] = jnp.exp(lr)
            return 0
        lax.fori_loop(0, CHUNK // 2, wcomp, 0)
        pltpu.async_copy(wb, den_sh.at[sidx], s_de, add=True)
        pltpu.make_async_copy(ht_hbm.at[gsrc], hb, s_ht).wait()

        def mul(i, _):
            for t in range(2):
                i2 = 2 * i + t
                wv = wb[i2, pl.ds(0, L)]
                for j in range(DC // L):
                    hb[i2, pl.ds(j * L, L)] = hb[i2, pl.ds(j * L, L)] * wv
            return 0
        lax.fori_loop(0, CHUNK // 2, mul, 0)
        pltpu.async_copy(hb, acc_sh.at[sidx], s_ac, add=True)

    def wait_scatters(sl):
        sidx, wb, hb = sl[4], sl[7], sl[8]
        s_de, s_ac = sl[14], sl[15]
        pltpu.make_async_copy(wb, den_sh.at[sidx], s_de).wait()
        pltpu.make_async_copy(hb, acc_sh.at[sidx], s_ac).wait()

    def process(a, sl, other):
        wait_eled(sl)

        @pl.when(a + 2 < NCHUNK)
        def _():
            issue_edges(a + 2, sl)
        wait_edges(a + 1, other)

        @pl.when(a >= 1)
        def _():
            wait_scatters(other)
        prep_idx(other)
        issue_gathers(other)
        compute_and_scatter(sl)

    issue_edges(0, slot0)

    def zero_hb(i, _):
        for j in range(DC // L):
            hb1[i, pl.ds(j * L, L)] = zf
        wb1[i, pl.ds(0, L)] = zf
        return 0
    lax.fori_loop(0, CHUNK, zero_hb, 0)

    wait_edges(0, slot0)
    prep_idx(slot0)
    issue_gathers(slot0)
    issue_edges(1, slot1)

    for k in range(STRIPE // CHUNK):
        pltpu.sync_copy(hb1, acc_sh.at[pl.ds(rbase + k * CHUNK, CHUNK)])
        pltpu.sync_copy(wb1, den_sh.at[pl.ds(rbase + k * CHUNK, CHUNK)])
    rem = STRIPE % CHUNK
    if rem:
        done = rbase + (STRIPE // CHUNK) * CHUNK
        pltpu.sync_copy(hb1.at[pl.ds(0, rem)], acc_sh.at[pl.ds(done, rem)])
        pltpu.sync_copy(wb1.at[pl.ds(0, rem)], den_sh.at[pl.ds(done, rem)])
    plsc.subcore_barrier()

    def pair(i, _):
        process(2 * i, slot0, slot1)
        process(2 * i + 1, slot1, slot0)
        return 0
    lax.fori_loop(0, (NCHUNK - 1) // 2, pair, 0)

    wait_eled(slot0)
    wait_scatters(slot1)
    compute_and_scatter(slot0)
    wait_scatters(slot0)

    plsc.subcore_barrier()
    gbase = cid * N + rbase

    @pl.when(sid < NS - 1)
    def _():
        pltpu.sync_copy(acc_sh.at[pl.ds(rbase, STRIPE)],
                        acc_out.at[pl.ds(gbase, STRIPE)])
        pltpu.sync_copy(den_sh.at[pl.ds(rbase, STRIPE)],
                        den_out.at[pl.ds(gbase, STRIPE)])

    @pl.when(sid == NS - 1)
    def _():
        pltpu.sync_copy(acc_sh.at[pl.ds(rbase, TAIL)],
                        acc_out.at[pl.ds(gbase, TAIL)])
        pltpu.sync_copy(den_sh.at[pl.ds(rbase, TAIL)],
                        den_out.at[pl.ds(gbase, TAIL)])


def _gat_sc(ht, esd, edd, e_src, e_dst):
    mesh = plsc.VectorSubcoreMesh(core_axis_name="c", subcore_axis_name="s",
                                  num_cores=NC, num_subcores=NS)
    fn = pl.kernel(
        _gat_sc_body,
        out_type=[
            jax.ShapeDtypeStruct((NC * N, DC), jnp.float32),
            jax.ShapeDtypeStruct((NC * N, L), jnp.float32),
        ],
        mesh=mesh,
        compiler_params=pltpu.CompilerParams(use_tc_tiling_on_sc=False),
        scratch_types=(
            [pltpu.VMEM((CHUNK,), jnp.int32) for _ in range(10)]
            + [pltpu.VMEM((CHUNK, L), jnp.float32) for _ in range(6)]
            + [pltpu.VMEM((CHUNK, DC), jnp.float32) for _ in range(2)]
            + [pltpu.VMEM_SHARED((RP, DC), jnp.float32),
               pltpu.VMEM_SHARED((RP, L), jnp.float32)]
            + [pltpu.SemaphoreType.DMA for _ in range(14)]
        ),
    )
    return fn(ht, esd, edd, e_src, e_dst)


def _post_body(x_ref, a0_ref, a1_ref, d0_ref, d1_ref, bt_ref, p_ref,
               g_ref, b2_ref, o_ref):
    den0 = jnp.concatenate([d0_ref[...]] * (DC // L), axis=1)
    den1 = jnp.concatenate([d1_ref[...]] * (DC // L), axis=1)
    y0 = jnp.maximum(a0_ref[...] / (den0 + 1e-16) + bt_ref[0][None], 0.0)
    y1 = jnp.maximum(a1_ref[...] / (den1 + 1e-16) + bt_ref[1][None], 0.0)
    z = (jnp.dot(y0, p_ref[0], preferred_element_type=jnp.float32)
         + jnp.dot(y1, p_ref[1], preferred_element_type=jnp.float32))
    o = x_ref[...] + z
    mu = jnp.mean(o, axis=-1, keepdims=True)
    var = jnp.mean((o - mu) ** 2, axis=-1, keepdims=True)
    o = (o - mu) * lax.rsqrt(var + 1e-6)
    o_ref[...] = o * g_ref[...] + b2_ref[...]


def _post(x, acc, den, bt, P, gamma, beta):
    return pl.pallas_call(
        _post_body,
        grid=(GB,),
        in_specs=[
            pl.BlockSpec((BN, D), lambda i: (i, 0)),
            pl.BlockSpec((BN, DC), lambda i: (i, 0)),
            pl.BlockSpec((BN, DC), lambda i: (GB + i, 0)),
            pl.BlockSpec((BN, L), lambda i: (i, 0)),
            pl.BlockSpec((BN, L), lambda i: (GB + i, 0)),
            pl.BlockSpec((NC, DC), lambda i: (0, 0)),
            pl.BlockSpec((NC, DC, D), lambda i: (0, 0, 0)),
            pl.BlockSpec((1, D), lambda i: (0, 0)),
            pl.BlockSpec((1, D), lambda i: (0, 0)),
        ],
        out_specs=pl.BlockSpec((BN, D), lambda i: (i, 0)),
        out_shape=jax.ShapeDtypeStruct((N, D), jnp.float32),
    )(x, acc, acc, den, den, bt, P, gamma, beta)


def kernel(x, W, a_src, a_dst, b, gamma, beta, edge_index):
    r = jnp.arange(DC)
    hh = r % HC
    dd = r // HC
    lane = jnp.arange(L) % HC
    msk = (lane[None, :] == hh[:, None])

    Wps, Ass, Ads, bts, Ps = [], [], [], [], []
    for c in range(NC):
        heads = c * HC + hh
        perm = heads * DH + dd
        Wps.append(W[:, perm])
        Ass.append(jnp.where(msk, a_src[heads, dd][:, None], 0.0))
        Ads.append(jnp.where(msk, a_dst[heads, dd][:, None], 0.0))
        bts.append(b[perm])
        Ps.append(jnp.zeros((DC, D), jnp.float32).at[r, perm].set(1.0))
    Wp = jnp.stack(Wps)
    As = jnp.stack(Ass)
    Ad = jnp.stack(Ads)
    bt = jnp.stack(bts)
    P = jnp.stack(Ps)

    npad = EPAD - E
    e_src = jnp.concatenate([edge_index[0], jnp.zeros((npad,), jnp.int32)])
    e_dst = jnp.concatenate([edge_index[1],
                             jnp.full((npad,), TRASH_DST, jnp.int32)])

    ht, esd, edd = _project(x, Wp, As, Ad)
    acc, den = _gat_sc(ht, esd, edd, e_src, e_dst)
    return _post(x, acc, den, bt, P, gamma[None], beta[None])


# speedup vs baseline: 1.0074x; 1.0074x over previous
import jax
import jax.numpy as jnp
from jax import lax
from jax.experimental import pallas as pl
from jax.experimental.pallas import tpu as pltpu
from jax.experimental.pallas import tpu_sc as plsc

N = 10000
E = 160000
D = 256
H = 8
DH = D // H

NC = 2
NS = 16
L = 16
HC = H // NC
DC = D // NC

RP = 10016
STRIPE = RP // NS
TAIL = N - (NS - 1) * STRIPE
CHUNK = 80
EPT = 10000
NCHUNK = EPT // CHUNK
EPAD = EPT * NS
TRASH_DST = 10008

BN = 1000
GB = N // BN


def _proj_body(x_ref, wp_ref, as_ref, ad_ref, ht_ref, es_ref, ed_ref):
    ht = jnp.dot(x_ref[...], wp_ref[0], preferred_element_type=jnp.float32)
    ht_ref[...] = ht
    es_ref[...] = jnp.dot(ht, as_ref[0], preferred_element_type=jnp.float32)
    ed_ref[...] = jnp.dot(ht, ad_ref[0], preferred_element_type=jnp.float32)


def _project(x, Wp, As, Ad):
    return pl.pallas_call(
        _proj_body,
        grid=(NC, GB),
        in_specs=[
            pl.BlockSpec((BN, D), lambda c, i: (i, 0)),
            pl.BlockSpec((1, D, DC), lambda c, i: (c, 0, 0)),
            pl.BlockSpec((1, DC, L), lambda c, i: (c, 0, 0)),
            pl.BlockSpec((1, DC, L), lambda c, i: (c, 0, 0)),
        ],
        out_specs=[
            pl.BlockSpec((BN, DC), lambda c, i: (c * GB + i, 0)),
            pl.BlockSpec((BN, L), lambda c, i: (c * GB + i, 0)),
            pl.BlockSpec((BN, L), lambda c, i: (c * GB + i, 0)),
        ],
        out_shape=[
            jax.ShapeDtypeStruct((NC * N, DC), jnp.float32),
            jax.ShapeDtypeStruct((NC * N, L), jnp.float32),
            jax.ShapeDtypeStruct((NC * N, L), jnp.float32),
        ],
    )(x, Wp, As, Ad)


def _gat_sc_body(ht_hbm, es_hbm, ed_hbm, esrc_hbm, edst_hbm, acc_out, den_out,
                 csrc0, csrc1, cdst0, cdst1, gsrc0, gsrc1, gdst0, gdst1,
                 sidx0, sidx1, esb0, esb1, edb0, edb1, wb0, wb1, hb0, hb1,
                 acc_sh, den_sh,
                 s_src0, s_src1, s_dst0, s_dst1, s_es0, s_es1, s_ed0, s_ed1,
                 s_ht0, s_ht1, s_de0, s_de1, s_ac0, s_ac1):
    cid = lax.axis_index("c")
    sid = lax.axis_index("s")
    base_t = (cid * N).astype(jnp.int32)
    base_tv = jnp.broadcast_to(base_t, (L,))
    nclampv = jnp.full((L,), N - 1, jnp.int32)
    zf = jnp.zeros((L,), jnp.float32)
    ebase = sid * EPT

    slot0 = (csrc0, cdst0, gsrc0, gdst0, sidx0, esb0, edb0, wb0, hb0,
             s_src0, s_dst0, s_es0, s_ed0, s_ht0, s_de0, s_ac0)
    slot1 = (csrc1, cdst1, gsrc1, gdst1, sidx1, esb1, edb1, wb1, hb1,
             s_src1, s_dst1, s_es1, s_ed1, s_ht1, s_de1, s_ac1)

    def zero_hb(i, _):
        for j in range(DC // L):
            hb0[i, pl.ds(j * L, L)] = zf
        wb0[i, pl.ds(0, L)] = zf
        return 0
    lax.fori_loop(0, CHUNK, zero_hb, 0)

    rbase = sid * STRIPE
    for k in range(STRIPE // CHUNK):
        pltpu.sync_copy(hb0, acc_sh.at[pl.ds(rbase + k * CHUNK, CHUNK)])
        pltpu.sync_copy(wb0, den_sh.at[pl.ds(rbase + k * CHUNK, CHUNK)])
    rem = STRIPE % CHUNK
    if rem:
        done = rbase + (STRIPE // CHUNK) * CHUNK
        pltpu.sync_copy(hb0.at[pl.ds(0, rem)], acc_sh.at[pl.ds(done, rem)])
        pltpu.sync_copy(wb0.at[pl.ds(0, rem)], den_sh.at[pl.ds(done, rem)])
    plsc.subcore_barrier()

    def issue_edges(k, sl):
        csrc, cdst = sl[0], sl[1]
        s_src, s_dst = sl[9], sl[10]
        eb = ebase + k * CHUNK
        pltpu.async_copy(esrc_hbm.at[pl.ds(eb, CHUNK)], csrc, s_src)
        pltpu.async_copy(edst_hbm.at[pl.ds(eb, CHUNK)], cdst, s_dst)

    def wait_edges(k, sl):
        csrc, cdst = sl[0], sl[1]
        s_src, s_dst = sl[9], sl[10]
        eb = ebase + k * CHUNK
        pltpu.make_async_copy(esrc_hbm.at[pl.ds(eb, CHUNK)], csrc, s_src).wait()
        pltpu.make_async_copy(edst_hbm.at[pl.ds(eb, CHUNK)], cdst, s_dst).wait()

    def prep_idx(sl):
        csrc, cdst, gsrc, gdst, sidx = sl[0], sl[1], sl[2], sl[3], sl[4]
        for q in range(CHUNK // L):
            sv = csrc[pl.ds(q * L, L)]
            gsrc[pl.ds(q * L, L)] = sv + base_tv
            dv = cdst[pl.ds(q * L, L)]
            sidx[pl.ds(q * L, L)] = dv
            gdst[pl.ds(q * L, L)] = jnp.minimum(dv, nclampv) + base_tv

    def issue_gathers(sl):
        gsrc, gdst, esb, edb, hb = sl[2], sl[3], sl[5], sl[6], sl[8]
        s_es, s_ed, s_ht = sl[11], sl[12], sl[13]
        pltpu.async_copy(es_hbm.at[gsrc], esb, s_es)
        pltpu.async_copy(ed_hbm.at[gdst], edb, s_ed)
        pltpu.async_copy(ht_hbm.at[gsrc], hb, s_ht)

    def wait_eled(sl):
        gsrc, gdst, esb, edb = sl[2], sl[3], sl[5], sl[6]
        s_es, s_ed = sl[11], sl[12]
        pltpu.make_async_copy(es_hbm.at[gsrc], esb, s_es).wait()
        pltpu.make_async_copy(ed_hbm.at[gdst], edb, s_ed).wait()

    def compute_and_scatter(sl):
        gsrc, sidx, esb, edb, wb, hb = sl[2], sl[4], sl[5], sl[6], sl[7], sl[8]
        s_ht, s_de, s_ac = sl[13], sl[14], sl[15]

        def wcomp(i, _):
            for t in range(2):
                i2 = 2 * i + t
                s = esb[i2, pl.ds(0, L)] + edb[i2, pl.ds(0, L)]
                lr = jnp.where(s > 0, s, 0.2 * s)
                wb[i2, pl.ds(0, L)] = jnp.exp(lr)
            return 0
        lax.fori_loop(0, CHUNK // 2, wcomp, 0)
        pltpu.async_copy(wb, den_sh.at[sidx], s_de, add=True)
        pltpu.make_async_copy(ht_hbm.at[gsrc], hb, s_ht).wait()

        def mul(i, _):
            for t in range(2):
                i2 = 2 * i + t
                wv = wb[i2, pl.ds(0, L)]
                for j in range(DC // L):
                    hb[i2, pl.ds(j * L, L)] = hb[i2, pl.ds(j * L, L)] * wv
            return 0
        lax.fori_loop(0, CHUNK // 2, mul, 0)
        pltpu.async_copy(hb, acc_sh.at[sidx], s_ac, add=True)

    def wait_scatters(sl):
        sidx, wb, hb = sl[4], sl[7], sl[8]
        s_de, s_ac = sl[14], sl[15]
        pltpu.make_async_copy(wb, den_sh.at[sidx], s_de).wait()
        pltpu.make_async_copy(hb, acc_sh.at[sidx], s_ac).wait()

    def process(a, sl, other):
        wait_eled(sl)

        @pl.when(a + 2 < NCHUNK)
        def _():
            issue_edges(a + 2, sl)
        wait_edges(a + 1, other)

        @pl.when(a >= 1)
        def _():
            wait_scatters(other)
        prep_idx(other)
        issue_gathers(other)
        compute_and_scatter(sl)

    issue_edges(0, slot0)
    wait_edges(0, slot0)
    prep_idx(slot0)
    issue_gathers(slot0)
    issue_edges(1, slot1)

    def pair(i, _):
        process(2 * i, slot0, slot1)
        process(2 * i + 1, slot1, slot0)
        return 0
    lax.fori_loop(0, (NCHUNK - 1) // 2, pair, 0)

    wait_eled(slot0)
    wait_scatters(slot1)
    compute_and_scatter(slot0)
    wait_scatters(slot0)

    plsc.subcore_barrier()
    gbase = cid * N + rbase

    @pl.when(sid < NS - 1)
    def _():
        pltpu.sync_copy(acc_sh.at[pl.ds(rbase, STRIPE)],
                        acc_out.at[pl.ds(gbase, STRIPE)])
        pltpu.sync_copy(den_sh.at[pl.ds(rbase, STRIPE)],
                        den_out.at[pl.ds(gbase, STRIPE)])

    @pl.when(sid == NS - 1)
    def _():
        pltpu.sync_copy(acc_sh.at[pl.ds(rbase, TAIL)],
                        acc_out.at[pl.ds(gbase, TAIL)])
        pltpu.sync_copy(den_sh.at[pl.ds(rbase, TAIL)],
                        den_out.at[pl.ds(gbase, TAIL)])


def _gat_sc(ht, esd, edd, e_src, e_dst):
    mesh = plsc.VectorSubcoreMesh(core_axis_name="c", subcore_axis_name="s",
                                  num_cores=NC, num_subcores=NS)
    fn = pl.kernel(
        _gat_sc_body,
        out_type=[
            jax.ShapeDtypeStruct((NC * N, DC), jnp.float32),
            jax.ShapeDtypeStruct((NC * N, L), jnp.float32),
        ],
        mesh=mesh,
        compiler_params=pltpu.CompilerParams(use_tc_tiling_on_sc=False),
        scratch_types=(
            [pltpu.VMEM((CHUNK,), jnp.int32) for _ in range(10)]
            + [pltpu.VMEM((CHUNK, L), jnp.float32) for _ in range(6)]
            + [pltpu.VMEM((CHUNK, DC), jnp.float32) for _ in range(2)]
            + [pltpu.VMEM_SHARED((RP, DC), jnp.float32),
               pltpu.VMEM_SHARED((RP, L), jnp.float32)]
            + [pltpu.SemaphoreType.DMA for _ in range(14)]
        ),
    )
    return fn(ht, esd, edd, e_src, e_dst)


def _post_body(x_ref, a0_ref, a1_ref, d0_ref, d1_ref, bt_ref, p_ref,
               g_ref, b2_ref, o_ref):
    den0 = jnp.concatenate([d0_ref[...]] * (DC // L), axis=1)
    den1 = jnp.concatenate([d1_ref[...]] * (DC // L), axis=1)
    y0 = jnp.maximum(a0_ref[...] / (den0 + 1e-16) + bt_ref[0][None], 0.0)
    y1 = jnp.maximum(a1_ref[...] / (den1 + 1e-16) + bt_ref[1][None], 0.0)
    z = (jnp.dot(y0, p_ref[0], preferred_element_type=jnp.float32)
         + jnp.dot(y1, p_ref[1], preferred_element_type=jnp.float32))
    o = x_ref[...] + z
    mu = jnp.mean(o, axis=-1, keepdims=True)
    var = jnp.mean((o - mu) ** 2, axis=-1, keepdims=True)
    o = (o - mu) * lax.rsqrt(var + 1e-6)
    o_ref[...] = o * g_ref[...] + b2_ref[...]


def _post(x, acc, den, bt, P, gamma, beta):
    return pl.pallas_call(
        _post_body,
        grid=(GB,),
        in_specs=[
            pl.BlockSpec((BN, D), lambda i: (i, 0)),
            pl.BlockSpec((BN, DC), lambda i: (i, 0)),
            pl.BlockSpec((BN, DC), lambda i: (GB + i, 0)),
            pl.BlockSpec((BN, L), lambda i: (i, 0)),
            pl.BlockSpec((BN, L), lambda i: (GB + i, 0)),
            pl.BlockSpec((NC, DC), lambda i: (0, 0)),
            pl.BlockSpec((NC, DC, D), lambda i: (0, 0, 0)),
            pl.BlockSpec((1, D), lambda i: (0, 0)),
            pl.BlockSpec((1, D), lambda i: (0, 0)),
        ],
        out_specs=pl.BlockSpec((BN, D), lambda i: (i, 0)),
        out_shape=jax.ShapeDtypeStruct((N, D), jnp.float32),
    )(x, acc, acc, den, den, bt, P, gamma, beta)


def kernel(x, W, a_src, a_dst, b, gamma, beta, edge_index):
    r = jnp.arange(DC)
    hh = r % HC
    dd = r // HC
    lane = jnp.arange(L) % HC
    msk = (lane[None, :] == hh[:, None])

    Wps, Ass, Ads, bts, Ps = [], [], [], [], []
    for c in range(NC):
        heads = c * HC + hh
        perm = heads * DH + dd
        Wps.append(W[:, perm])
        Ass.append(jnp.where(msk, a_src[heads, dd][:, None], 0.0))
        Ads.append(jnp.where(msk, a_dst[heads, dd][:, None], 0.0))
        bts.append(b[perm])
        Ps.append(jnp.zeros((DC, D), jnp.float32).at[r, perm].set(1.0))
    Wp = jnp.stack(Wps)
    As = jnp.stack(Ass)
    Ad = jnp.stack(Ads)
    bt = jnp.stack(bts)
    P = jnp.stack(Ps)

    npad = EPAD - E
    e_src = jnp.concatenate([edge_index[0], jnp.zeros((npad,), jnp.int32)])
    e_dst = jnp.concatenate([edge_index[1],
                             jnp.full((npad,), TRASH_DST, jnp.int32)])

    ht, esd, edd = _project(x, Wp, As, Ad)
    acc, den = _gat_sc(ht, esd, edd, e_src, e_dst)
    return _post(x, acc, den, bt, P, gamma[None], beta[None])
